# baseline (speedup 1.0000x reference)
import functools

import jax
import jax.numpy as jnp
from jax.experimental import pallas as pl
from jax.experimental.pallas import tpu as pltpu


def _copy_body(w_hbm, o_hbm, w_vmem, in_sem, out_sem, *, B, K, CT):
    ins = [
        pltpu.make_async_copy(
            w_hbm.at[pl.ds(k * CT, CT), :],
            w_vmem.at[pl.ds(k * CT, CT), :],
            in_sem.at[k],
        )
        for k in range(K)
    ]
    for c in ins:
        c.start()
    outs = []
    for k in range(K):
        ins[k].wait()
        for b in range(B):
            c = pltpu.make_async_copy(
                w_vmem.at[pl.ds(k * CT, CT), :],
                o_hbm.at[b, pl.ds(k * CT, CT), :],
                out_sem.at[b],
            )
            c.start()
            outs.append(c)
    for c in outs:
        c.wait()


def kernel(x, W):
    B = x.shape[0]
    T, H = W.shape
    K = 2
    CT = T // K
    body = functools.partial(_copy_body, B=B, K=K, CT=CT)
    return pl.pallas_call(
        body,
        in_specs=[pl.BlockSpec(memory_space=pl.ANY)],
        out_specs=pl.BlockSpec(memory_space=pl.ANY),
        out_shape=jax.ShapeDtypeStruct((B, T, H), W.dtype),
        scratch_shapes=[
            pltpu.VMEM((T, H), W.dtype),
            pltpu.SemaphoreType.DMA((K,)),
            pltpu.SemaphoreType.DMA((B,)),
        ],
    )(W)


# speedup vs baseline: 1.0261x; 1.0261x over previous
import functools

import jax
import jax.numpy as jnp
from jax.experimental import pallas as pl
from jax.experimental.pallas import tpu as pltpu


def _copy_body(w_hbm, o_hbm, w_vmem, in_sem, out_sem, *, B, K, CT):
    ins = [
        pltpu.make_async_copy(
            w_hbm.at[pl.ds(k * CT, CT), :],
            w_vmem.at[pl.ds(k * CT, CT), :],
            in_sem.at[k],
        )
        for k in range(K)
    ]
    for c in ins:
        c.start()
    outs = []
    for k in range(K):
        ins[k].wait()
        for b in range(B):
            c = pltpu.make_async_copy(
                w_vmem.at[pl.ds(k * CT, CT), :],
                o_hbm.at[b, pl.ds(k * CT, CT), :],
                out_sem.at[b],
            )
            c.start()
            outs.append(c)
    for c in outs:
        c.wait()


def kernel(x, W):
    B = x.shape[0]
    T, H = W.shape
    K = 4
    CT = T // K
    body = functools.partial(_copy_body, B=B, K=K, CT=CT)
    return pl.pallas_call(
        body,
        in_specs=[pl.BlockSpec(memory_space=pl.ANY)],
        out_specs=pl.BlockSpec(memory_space=pl.ANY),
        out_shape=jax.ShapeDtypeStruct((B, T, H), W.dtype),
        scratch_shapes=[
            pltpu.VMEM((T, H), W.dtype),
            pltpu.SemaphoreType.DMA((K,)),
            pltpu.SemaphoreType.DMA((B,)),
        ],
    )(W)
